# baseline (speedup 1.0000x reference)
import functools

import jax
import jax.numpy as jnp
from jax import lax
from jax.experimental import pallas as pl
from jax.experimental.pallas import tpu as pltpu
from jax.experimental.pallas import tpu_sc as plsc

_VB = 4096
_CHUNK = 400
_NBUF = 2


@functools.lru_cache(maxsize=None)
def _make_depad_transpose(vocab: int, d: int):
    grid = (vocab + _VB - 1) // _VB

    def body(t_ref, o_ref):
        y = t_ref[...].T
        o_ref[...] = jnp.concatenate([y, y], axis=1)

    return pl.pallas_call(
        body,
        grid=(grid,),
        in_specs=[pl.BlockSpec((d, _VB), lambda g: (0, g))],
        out_specs=pl.BlockSpec((_VB, 2 * d), lambda g: (g, 0)),
        out_shape=jax.ShapeDtypeStruct((vocab, 2 * d), jnp.float32),
    )


@functools.lru_cache(maxsize=None)
def _make_gather(n_flat: int, vocab: int, d: int):
    info = plsc.get_sparse_core_info()
    nw = info.num_cores * info.num_subcores
    assert n_flat % (nw * _CHUNK * _NBUF) == 0 and _CHUNK % 8 == 0
    b_per_w = n_flat // nw
    n_chunks = b_per_w // _CHUNK
    n_steps = n_chunks // _NBUF
    mesh = plsc.VectorSubcoreMesh(core_axis_name="c", subcore_axis_name="s")

    @functools.partial(
        pl.kernel,
        mesh=mesh,
        out_type=jax.ShapeDtypeStruct((n_flat // 8, 8, 2 * d), jnp.float32),
        scratch_types=[
            pltpu.VMEM((_NBUF, _CHUNK), jnp.int32),
            pltpu.VMEM((_NBUF, _CHUNK, d), jnp.float32),
            pltpu.SemaphoreType.DMA((_NBUF,)),
            pltpu.SemaphoreType.DMA((_NBUF,)),
        ],
        compiler_params=pltpu.CompilerParams(use_tc_tiling_on_sc=False),
    )
    def gather_kernel(idx_hbm, table_hbm, out_hbm, idx_v, rows_v, gsem, ssem):
        wid = lax.axis_index("s") * info.num_cores + lax.axis_index("c")
        base = wid * b_per_w

        def start_gather(ci, b):
            row0 = base + ci * _CHUNK
            pltpu.sync_copy(idx_hbm.at[pl.ds(row0, _CHUNK)], idx_v.at[b])
            pltpu.async_copy(table_hbm.at[idx_v.at[b]], rows_v.at[b],
                             gsem.at[b])

        def start_scatter(ci, b):
            g0 = (base + ci * _CHUNK) // 8
            for r in range(_CHUNK // 8):
                pltpu.async_copy(
                    rows_v.at[b, pl.ds(r * 8, 8)],
                    out_hbm.at[g0 + r, slice(None), pl.ds(0, d)],
                    ssem.at[b],
                )

        def wait_gather(ci, b):
            pltpu.make_async_copy(
                table_hbm.at[idx_v.at[b]], rows_v.at[b], gsem.at[b]
            ).wait()

        def wait_scatter(ci, b):
            g0 = (base + ci * _CHUNK) // 8
            for r in range(_CHUNK // 8):
                pltpu.make_async_copy(
                    rows_v.at[b, pl.ds(r * 8, 8)],
                    out_hbm.at[g0 + r, slice(None), pl.ds(0, d)],
                    ssem.at[b],
                ).wait()

        for b in range(_NBUF):
            start_gather(b, b)

        def body(g, carry):
            for b in range(_NBUF):
                ci = g * _NBUF + b
                wait_gather(ci, b)
                start_scatter(ci, b)
                wait_scatter(ci, b)
                start_gather(ci + _NBUF, b)
            return carry

        lax.fori_loop(0, n_steps - 1, body, 0)

        for b in range(_NBUF):
            ci = (n_steps - 1) * _NBUF + b
            wait_gather(ci, b)
            start_scatter(ci, b)
        for b in range(_NBUF):
            ci = (n_steps - 1) * _NBUF + b
            wait_scatter(ci, b)

    return gather_kernel


def kernel(input, table):
    b, l = input.shape
    vocab, d = table.shape
    n = b * l
    td = _make_depad_transpose(vocab, d)(table.T)
    flat_idx = input.T.reshape(n) * 2
    packed = _make_gather(n, vocab, d)(flat_idx, td.reshape(2 * vocab, d))
    emb = packed.reshape(n, 2 * d)[:, :d].reshape(l, b, d)
    return emb.transpose(1, 0, 2)


# speedup vs baseline: 1.2102x; 1.2102x over previous
import functools

import jax
import jax.numpy as jnp
from jax import lax
from jax.experimental import pallas as pl
from jax.experimental.pallas import tpu as pltpu
from jax.experimental.pallas import tpu_sc as plsc

_VB = 8192
_CHUNK = 400
_NBUF = 2


@functools.lru_cache(maxsize=None)
def _make_depad_transpose(vocab: int, d: int):
    grid = (vocab + _VB - 1) // _VB

    def body(t_ref, o_ref):
        o_ref[:, :d] = t_ref[...].T

    return pl.pallas_call(
        body,
        grid=(grid,),
        in_specs=[pl.BlockSpec((d, _VB), lambda g: (0, g))],
        out_specs=pl.BlockSpec((_VB, 2 * d), lambda g: (g, 0)),
        out_shape=jax.ShapeDtypeStruct((vocab, 2 * d), jnp.float32),
    )


@functools.lru_cache(maxsize=None)
def _make_gather(n_flat: int, vocab: int, d: int):
    info = plsc.get_sparse_core_info()
    nw = info.num_cores * info.num_subcores
    assert n_flat % (nw * _CHUNK * _NBUF) == 0 and _CHUNK % 8 == 0
    b_per_w = n_flat // nw
    n_chunks = b_per_w // _CHUNK
    n_steps = n_chunks // _NBUF
    mesh = plsc.VectorSubcoreMesh(core_axis_name="c", subcore_axis_name="s")

    @functools.partial(
        pl.kernel,
        mesh=mesh,
        out_type=jax.ShapeDtypeStruct((n_flat // 8, 8, 2 * d), jnp.float32),
        scratch_types=[
            pltpu.VMEM((_NBUF, _CHUNK), jnp.int32),
            pltpu.VMEM((_NBUF, _CHUNK, d), jnp.float32),
            pltpu.SemaphoreType.DMA((_NBUF,)),
            pltpu.SemaphoreType.DMA((_NBUF,)),
        ],
        compiler_params=pltpu.CompilerParams(use_tc_tiling_on_sc=False),
    )
    def gather_kernel(idx_hbm, table_hbm, out_hbm, idx_v, rows_v, gsem, ssem):
        wid = lax.axis_index("s") * info.num_cores + lax.axis_index("c")
        base = wid * b_per_w

        def start_gather(ci, b):
            row0 = base + ci * _CHUNK
            pltpu.sync_copy(idx_hbm.at[pl.ds(row0, _CHUNK)], idx_v.at[b])
            pltpu.async_copy(table_hbm.at[idx_v.at[b]], rows_v.at[b],
                             gsem.at[b])

        def start_scatter(ci, b):
            g0 = (base + ci * _CHUNK) // 8
            for r in range(_CHUNK // 8):
                pltpu.async_copy(
                    rows_v.at[b, pl.ds(r * 8, 8)],
                    out_hbm.at[g0 + r, slice(None), pl.ds(0, d)],
                    ssem.at[b],
                )

        def wait_gather(ci, b):
            pltpu.make_async_copy(
                table_hbm.at[idx_v.at[b]], rows_v.at[b], gsem.at[b]
            ).wait()

        def wait_scatter(ci, b):
            g0 = (base + ci * _CHUNK) // 8
            for r in range(_CHUNK // 8):
                pltpu.make_async_copy(
                    rows_v.at[b, pl.ds(r * 8, 8)],
                    out_hbm.at[g0 + r, slice(None), pl.ds(0, d)],
                    ssem.at[b],
                ).wait()

        for b in range(_NBUF):
            start_gather(b, b)

        def body(g, carry):
            for b in range(_NBUF):
                ci = g * _NBUF + b
                wait_gather(ci, b)
                start_scatter(ci, b)
                wait_scatter(ci, b)
                start_gather(ci + _NBUF, b)
            return carry

        lax.fori_loop(0, n_steps - 1, body, 0)

        for b in range(_NBUF):
            ci = (n_steps - 1) * _NBUF + b
            wait_gather(ci, b)
            start_scatter(ci, b)
        for b in range(_NBUF):
            ci = (n_steps - 1) * _NBUF + b
            wait_scatter(ci, b)

    return gather_kernel


def kernel(input, table):
    b, l = input.shape
    vocab, d = table.shape
    n = b * l
    td = _make_depad_transpose(vocab, d)(table.T)
    flat_idx = input.T.reshape(n) * 2
    packed = _make_gather(n, vocab, d)(flat_idx, td.reshape(2 * vocab, d))
    emb = packed.reshape(n, 2 * d)[:, :d].reshape(l, b, d)
    return emb.transpose(1, 0, 2)


# speedup vs baseline: 1.2551x; 1.0371x over previous
import functools

import jax
import jax.numpy as jnp
from jax import lax
from jax.experimental import pallas as pl
from jax.experimental.pallas import tpu as pltpu
from jax.experimental.pallas import tpu_sc as plsc

_VB = 16384
_CHUNK = 400
_NBUF = 2


@functools.lru_cache(maxsize=None)
def _make_depad_transpose(vocab: int, d: int):
    grid = (vocab + _VB - 1) // _VB

    def body(t_ref, o_ref):
        o_ref[:, :d] = t_ref[...].T

    return pl.pallas_call(
        body,
        grid=(grid,),
        in_specs=[pl.BlockSpec((d, _VB), lambda g: (0, g))],
        out_specs=pl.BlockSpec((_VB, 2 * d), lambda g: (g, 0)),
        out_shape=jax.ShapeDtypeStruct((vocab, 2 * d), jnp.float32),
    )


@functools.lru_cache(maxsize=None)
def _make_gather(n_flat: int, vocab: int, d: int):
    info = plsc.get_sparse_core_info()
    nw = info.num_cores * info.num_subcores
    assert n_flat % (nw * _CHUNK * _NBUF) == 0 and _CHUNK % 8 == 0
    b_per_w = n_flat // nw
    n_chunks = b_per_w // _CHUNK
    n_steps = n_chunks // _NBUF
    mesh = plsc.VectorSubcoreMesh(core_axis_name="c", subcore_axis_name="s")

    @functools.partial(
        pl.kernel,
        mesh=mesh,
        out_type=jax.ShapeDtypeStruct((n_flat // 8, 8, 2 * d), jnp.float32),
        scratch_types=[
            pltpu.VMEM((_NBUF, _CHUNK), jnp.int32),
            pltpu.VMEM((_NBUF, _CHUNK, d), jnp.float32),
            pltpu.SemaphoreType.DMA((_NBUF,)),
            pltpu.SemaphoreType.DMA((_NBUF,)),
        ],
        compiler_params=pltpu.CompilerParams(use_tc_tiling_on_sc=False),
    )
    def gather_kernel(idx_hbm, table_hbm, out_hbm, idx_v, rows_v, gsem, ssem):
        wid = lax.axis_index("s") * info.num_cores + lax.axis_index("c")
        base = wid * b_per_w

        def start_gather(ci, b):
            row0 = base + ci * _CHUNK
            pltpu.sync_copy(idx_hbm.at[pl.ds(row0, _CHUNK)], idx_v.at[b])
            pltpu.async_copy(table_hbm.at[idx_v.at[b]], rows_v.at[b],
                             gsem.at[b])

        def start_scatter(ci, b):
            g0 = (base + ci * _CHUNK) // 8
            for r in range(_CHUNK // 8):
                pltpu.async_copy(
                    rows_v.at[b, pl.ds(r * 8, 8)],
                    out_hbm.at[g0 + r, slice(None), pl.ds(0, d)],
                    ssem.at[b],
                )

        def wait_gather(ci, b):
            pltpu.make_async_copy(
                table_hbm.at[idx_v.at[b]], rows_v.at[b], gsem.at[b]
            ).wait()

        def wait_scatter(ci, b):
            g0 = (base + ci * _CHUNK) // 8
            for r in range(_CHUNK // 8):
                pltpu.make_async_copy(
                    rows_v.at[b, pl.ds(r * 8, 8)],
                    out_hbm.at[g0 + r, slice(None), pl.ds(0, d)],
                    ssem.at[b],
                ).wait()

        for b in range(_NBUF):
            start_gather(b, b)

        def body(g, carry):
            for b in range(_NBUF):
                ci = g * _NBUF + b
                wait_gather(ci, b)
                start_scatter(ci, b)
                wait_scatter(ci, b)
                start_gather(ci + _NBUF, b)
            return carry

        lax.fori_loop(0, n_steps - 1, body, 0)

        for b in range(_NBUF):
            ci = (n_steps - 1) * _NBUF + b
            wait_gather(ci, b)
            start_scatter(ci, b)
        for b in range(_NBUF):
            ci = (n_steps - 1) * _NBUF + b
            wait_scatter(ci, b)

    return gather_kernel


def kernel(input, table):
    b, l = input.shape
    vocab, d = table.shape
    n = b * l
    td = _make_depad_transpose(vocab, d)(table.T)
    flat_idx = input.T.reshape(n) * 2
    packed = _make_gather(n, vocab, d)(flat_idx, td.reshape(2 * vocab, d))
    emb = packed.reshape(n, 2 * d)[:, :d].reshape(l, b, d)
    return emb.transpose(1, 0, 2)


# speedup vs baseline: 1.2668x; 1.0094x over previous
import functools

import jax
import jax.numpy as jnp
from jax import lax
from jax.experimental import pallas as pl
from jax.experimental.pallas import tpu as pltpu
from jax.experimental.pallas import tpu_sc as plsc

_VB = 32768
_CHUNK = 400
_NBUF = 2


@functools.lru_cache(maxsize=None)
def _make_depad_transpose(vocab: int, d: int):
    grid = (vocab + _VB - 1) // _VB

    def body(t_ref, o_ref):
        o_ref[:, :d] = t_ref[...].T

    return pl.pallas_call(
        body,
        grid=(grid,),
        in_specs=[pl.BlockSpec((d, _VB), lambda g: (0, g))],
        out_specs=pl.BlockSpec((_VB, 2 * d), lambda g: (g, 0)),
        out_shape=jax.ShapeDtypeStruct((vocab, 2 * d), jnp.float32),
    )


@functools.lru_cache(maxsize=None)
def _make_gather(n_flat: int, vocab: int, d: int):
    info = plsc.get_sparse_core_info()
    nw = info.num_cores * info.num_subcores
    assert n_flat % (nw * _CHUNK * _NBUF) == 0 and _CHUNK % 8 == 0
    b_per_w = n_flat // nw
    n_chunks = b_per_w // _CHUNK
    n_steps = n_chunks // _NBUF
    mesh = plsc.VectorSubcoreMesh(core_axis_name="c", subcore_axis_name="s")

    @functools.partial(
        pl.kernel,
        mesh=mesh,
        out_type=jax.ShapeDtypeStruct((n_flat // 8, 8, 2 * d), jnp.float32),
        scratch_types=[
            pltpu.VMEM((_NBUF, _CHUNK), jnp.int32),
            pltpu.VMEM((_NBUF, _CHUNK, d), jnp.float32),
            pltpu.SemaphoreType.DMA((_NBUF,)),
            pltpu.SemaphoreType.DMA((_NBUF,)),
        ],
        compiler_params=pltpu.CompilerParams(use_tc_tiling_on_sc=False),
    )
    def gather_kernel(idx_hbm, table_hbm, out_hbm, idx_v, rows_v, gsem, ssem):
        wid = lax.axis_index("s") * info.num_cores + lax.axis_index("c")
        base = wid * b_per_w

        def start_gather(ci, b):
            row0 = base + ci * _CHUNK
            pltpu.sync_copy(idx_hbm.at[pl.ds(row0, _CHUNK)], idx_v.at[b])
            pltpu.async_copy(table_hbm.at[idx_v.at[b]], rows_v.at[b],
                             gsem.at[b])

        def start_scatter(ci, b):
            g0 = (base + ci * _CHUNK) // 8
            for r in range(_CHUNK // 8):
                pltpu.async_copy(
                    rows_v.at[b, pl.ds(r * 8, 8)],
                    out_hbm.at[g0 + r, slice(None), pl.ds(0, d)],
                    ssem.at[b],
                )

        def wait_gather(ci, b):
            pltpu.make_async_copy(
                table_hbm.at[idx_v.at[b]], rows_v.at[b], gsem.at[b]
            ).wait()

        def wait_scatter(ci, b):
            g0 = (base + ci * _CHUNK) // 8
            for r in range(_CHUNK // 8):
                pltpu.make_async_copy(
                    rows_v.at[b, pl.ds(r * 8, 8)],
                    out_hbm.at[g0 + r, slice(None), pl.ds(0, d)],
                    ssem.at[b],
                ).wait()

        for b in range(_NBUF):
            start_gather(b, b)

        def body(g, carry):
            for b in range(_NBUF):
                ci = g * _NBUF + b
                wait_gather(ci, b)
                start_scatter(ci, b)
                wait_scatter(ci, b)
                start_gather(ci + _NBUF, b)
            return carry

        lax.fori_loop(0, n_steps - 1, body, 0)

        for b in range(_NBUF):
            ci = (n_steps - 1) * _NBUF + b
            wait_gather(ci, b)
            start_scatter(ci, b)
        for b in range(_NBUF):
            ci = (n_steps - 1) * _NBUF + b
            wait_scatter(ci, b)

    return gather_kernel


def kernel(input, table):
    b, l = input.shape
    vocab, d = table.shape
    n = b * l
    td = _make_depad_transpose(vocab, d)(table.T)
    flat_idx = input.T.reshape(n) * 2
    packed = _make_gather(n, vocab, d)(flat_idx, td.reshape(2 * vocab, d))
    emb = packed.reshape(n, 2 * d)[:, :d].reshape(l, b, d)
    return emb.transpose(1, 0, 2)


# speedup vs baseline: 1.2811x; 1.0112x over previous
import functools

import jax
import jax.numpy as jnp
from jax import lax
from jax.experimental import pallas as pl
from jax.experimental.pallas import tpu as pltpu
from jax.experimental.pallas import tpu_sc as plsc

_VB = 32768
_CHUNK = 512
_NBUF = 2


@functools.lru_cache(maxsize=None)
def _make_depad_transpose(vocab: int, d: int):
    grid = (vocab + _VB - 1) // _VB

    def body(t_ref, o_ref):
        o_ref[:, :d] = t_ref[...].T

    return pl.pallas_call(
        body,
        grid=(grid,),
        in_specs=[pl.BlockSpec((d, _VB), lambda g: (0, g))],
        out_specs=pl.BlockSpec((_VB, 2 * d), lambda g: (g, 0)),
        out_shape=jax.ShapeDtypeStruct((vocab, 2 * d), jnp.float32),
    )


@functools.lru_cache(maxsize=None)
def _make_gather(n_flat: int, vocab: int, d: int):
    info = plsc.get_sparse_core_info()
    nw = info.num_cores * info.num_subcores
    assert n_flat % (nw * _CHUNK * _NBUF) == 0 and _CHUNK % 8 == 0
    b_per_w = n_flat // nw
    n_chunks = b_per_w // _CHUNK
    n_steps = n_chunks // _NBUF
    mesh = plsc.VectorSubcoreMesh(core_axis_name="c", subcore_axis_name="s")

    @functools.partial(
        pl.kernel,
        mesh=mesh,
        out_type=jax.ShapeDtypeStruct((n_flat // 8, 8, 2 * d), jnp.float32),
        scratch_types=[
            pltpu.VMEM((_NBUF, _CHUNK), jnp.int32),
            pltpu.VMEM((_NBUF, _CHUNK, d), jnp.float32),
            pltpu.SemaphoreType.DMA((_NBUF,)),
            pltpu.SemaphoreType.DMA((_NBUF,)),
        ],
        compiler_params=pltpu.CompilerParams(use_tc_tiling_on_sc=False),
    )
    def gather_kernel(idx_hbm, table_hbm, out_hbm, idx_v, rows_v, gsem, ssem):
        wid = lax.axis_index("s") * info.num_cores + lax.axis_index("c")
        base = wid * b_per_w

        def start_gather(ci, b):
            row0 = base + ci * _CHUNK
            pltpu.sync_copy(idx_hbm.at[pl.ds(row0, _CHUNK)], idx_v.at[b])
            pltpu.async_copy(table_hbm.at[idx_v.at[b]], rows_v.at[b],
                             gsem.at[b])

        def start_scatter(ci, b):
            g0 = (base + ci * _CHUNK) // 8
            for r in range(_CHUNK // 8):
                pltpu.async_copy(
                    rows_v.at[b, pl.ds(r * 8, 8)],
                    out_hbm.at[g0 + r, slice(None), pl.ds(0, d)],
                    ssem.at[b],
                )

        def wait_gather(ci, b):
            pltpu.make_async_copy(
                table_hbm.at[idx_v.at[b]], rows_v.at[b], gsem.at[b]
            ).wait()

        def wait_scatter(ci, b):
            g0 = (base + ci * _CHUNK) // 8
            for r in range(_CHUNK // 8):
                pltpu.make_async_copy(
                    rows_v.at[b, pl.ds(r * 8, 8)],
                    out_hbm.at[g0 + r, slice(None), pl.ds(0, d)],
                    ssem.at[b],
                ).wait()

        for b in range(_NBUF):
            start_gather(b, b)

        def body(g, carry):
            for b in range(_NBUF):
                ci = g * _NBUF + b
                wait_gather(ci, b)
                start_scatter(ci, b)
                wait_scatter(ci, b)
                start_gather(ci + _NBUF, b)
            return carry

        lax.fori_loop(0, n_steps - 1, body, 0)

        for b in range(_NBUF):
            ci = (n_steps - 1) * _NBUF + b
            wait_gather(ci, b)
            start_scatter(ci, b)
        for b in range(_NBUF):
            ci = (n_steps - 1) * _NBUF + b
            wait_scatter(ci, b)

    return gather_kernel


def kernel(input, table):
    b, l = input.shape
    vocab, d = table.shape
    n = b * l
    td = _make_depad_transpose(vocab, d)(table.T)
    flat_idx = input.T.reshape(n) * 2
    packed = _make_gather(n, vocab, d)(flat_idx, td.reshape(2 * vocab, d))
    emb = packed.reshape(n, 2 * d)[:, :d].reshape(l, b, d)
    return emb.transpose(1, 0, 2)


# speedup vs baseline: 1.2865x; 1.0042x over previous
import functools

import jax
import jax.numpy as jnp
from jax import lax
from jax.experimental import pallas as pl
from jax.experimental.pallas import tpu as pltpu
from jax.experimental.pallas import tpu_sc as plsc

_VB = 32768
_CHUNK = 800
_NBUF = 2


@functools.lru_cache(maxsize=None)
def _make_depad_transpose(vocab: int, d: int):
    grid = (vocab + _VB - 1) // _VB

    def body(t_ref, o_ref):
        o_ref[:, :d] = t_ref[...].T

    return pl.pallas_call(
        body,
        grid=(grid,),
        in_specs=[pl.BlockSpec((d, _VB), lambda g: (0, g))],
        out_specs=pl.BlockSpec((_VB, 2 * d), lambda g: (g, 0)),
        out_shape=jax.ShapeDtypeStruct((vocab, 2 * d), jnp.float32),
    )


@functools.lru_cache(maxsize=None)
def _make_gather(n_flat: int, vocab: int, d: int):
    info = plsc.get_sparse_core_info()
    nw = info.num_cores * info.num_subcores
    assert n_flat % (nw * _CHUNK * _NBUF) == 0 and _CHUNK % 8 == 0
    b_per_w = n_flat // nw
    n_chunks = b_per_w // _CHUNK
    n_steps = n_chunks // _NBUF
    mesh = plsc.VectorSubcoreMesh(core_axis_name="c", subcore_axis_name="s")

    @functools.partial(
        pl.kernel,
        mesh=mesh,
        out_type=jax.ShapeDtypeStruct((n_flat // 8, 8, 2 * d), jnp.float32),
        scratch_types=[
            pltpu.VMEM((_NBUF, _CHUNK), jnp.int32),
            pltpu.VMEM((_NBUF, _CHUNK, d), jnp.float32),
            pltpu.SemaphoreType.DMA((_NBUF,)),
            pltpu.SemaphoreType.DMA((_NBUF,)),
        ],
        compiler_params=pltpu.CompilerParams(use_tc_tiling_on_sc=False),
    )
    def gather_kernel(idx_hbm, table_hbm, out_hbm, idx_v, rows_v, gsem, ssem):
        wid = lax.axis_index("s") * info.num_cores + lax.axis_index("c")
        base = wid * b_per_w

        def start_gather(ci, b):
            row0 = base + ci * _CHUNK
            pltpu.sync_copy(idx_hbm.at[pl.ds(row0, _CHUNK)], idx_v.at[b])
            pltpu.async_copy(table_hbm.at[idx_v.at[b]], rows_v.at[b],
                             gsem.at[b])

        def start_scatter(ci, b):
            g0 = (base + ci * _CHUNK) // 8
            for r in range(_CHUNK // 8):
                pltpu.async_copy(
                    rows_v.at[b, pl.ds(r * 8, 8)],
                    out_hbm.at[g0 + r, slice(None), pl.ds(0, d)],
                    ssem.at[b],
                )

        def wait_gather(ci, b):
            pltpu.make_async_copy(
                table_hbm.at[idx_v.at[b]], rows_v.at[b], gsem.at[b]
            ).wait()

        def wait_scatter(ci, b):
            g0 = (base + ci * _CHUNK) // 8
            for r in range(_CHUNK // 8):
                pltpu.make_async_copy(
                    rows_v.at[b, pl.ds(r * 8, 8)],
                    out_hbm.at[g0 + r, slice(None), pl.ds(0, d)],
                    ssem.at[b],
                ).wait()

        for b in range(_NBUF):
            start_gather(b, b)

        def body(g, carry):
            for b in range(_NBUF):
                ci = g * _NBUF + b
                wait_gather(ci, b)
                start_scatter(ci, b)
                wait_scatter(ci, b)
                start_gather(ci + _NBUF, b)
            return carry

        lax.fori_loop(0, n_steps - 1, body, 0)

        for b in range(_NBUF):
            ci = (n_steps - 1) * _NBUF + b
            wait_gather(ci, b)
            start_scatter(ci, b)
        for b in range(_NBUF):
            ci = (n_steps - 1) * _NBUF + b
            wait_scatter(ci, b)

    return gather_kernel


def kernel(input, table):
    b, l = input.shape
    vocab, d = table.shape
    n = b * l
    td = _make_depad_transpose(vocab, d)(table.T)
    flat_idx = input.T.reshape(n) * 2
    packed = _make_gather(n, vocab, d)(flat_idx, td.reshape(2 * vocab, d))
    emb = packed.reshape(n, 2 * d)[:, :d].reshape(l, b, d)
    return emb.transpose(1, 0, 2)


# speedup vs baseline: 1.2901x; 1.0028x over previous
import functools

import jax
import jax.numpy as jnp
from jax import lax
from jax.experimental import pallas as pl
from jax.experimental.pallas import tpu as pltpu
from jax.experimental.pallas import tpu_sc as plsc

_VB = 32768
_CHUNK = 400
_NBUF = 4


@functools.lru_cache(maxsize=None)
def _make_depad_transpose(vocab: int, d: int):
    grid = (vocab + _VB - 1) // _VB

    def body(t_ref, o_ref):
        o_ref[:, :d] = t_ref[...].T

    return pl.pallas_call(
        body,
        grid=(grid,),
        in_specs=[pl.BlockSpec((d, _VB), lambda g: (0, g))],
        out_specs=pl.BlockSpec((_VB, 2 * d), lambda g: (g, 0)),
        out_shape=jax.ShapeDtypeStruct((vocab, 2 * d), jnp.float32),
    )


@functools.lru_cache(maxsize=None)
def _make_gather(n_flat: int, vocab: int, d: int):
    info = plsc.get_sparse_core_info()
    nw = info.num_cores * info.num_subcores
    assert n_flat % (nw * _CHUNK * _NBUF) == 0 and _CHUNK % 8 == 0
    b_per_w = n_flat // nw
    n_chunks = b_per_w // _CHUNK
    n_steps = n_chunks // _NBUF
    mesh = plsc.VectorSubcoreMesh(core_axis_name="c", subcore_axis_name="s")

    @functools.partial(
        pl.kernel,
        mesh=mesh,
        out_type=jax.ShapeDtypeStruct((n_flat // 8, 8, 2 * d), jnp.float32),
        scratch_types=[
            pltpu.VMEM((_NBUF, _CHUNK), jnp.int32),
            pltpu.VMEM((_NBUF, _CHUNK, d), jnp.float32),
            pltpu.SemaphoreType.DMA((_NBUF,)),
            pltpu.SemaphoreType.DMA((_NBUF,)),
        ],
        compiler_params=pltpu.CompilerParams(use_tc_tiling_on_sc=False),
    )
    def gather_kernel(idx_hbm, table_hbm, out_hbm, idx_v, rows_v, gsem, ssem):
        wid = lax.axis_index("s") * info.num_cores + lax.axis_index("c")
        base = wid * b_per_w

        def start_gather(ci, b):
            row0 = base + ci * _CHUNK
            pltpu.sync_copy(idx_hbm.at[pl.ds(row0, _CHUNK)], idx_v.at[b])
            pltpu.async_copy(table_hbm.at[idx_v.at[b]], rows_v.at[b],
                             gsem.at[b])

        def start_scatter(ci, b):
            g0 = (base + ci * _CHUNK) // 8
            for r in range(_CHUNK // 8):
                pltpu.async_copy(
                    rows_v.at[b, pl.ds(r * 8, 8)],
                    out_hbm.at[g0 + r, slice(None), pl.ds(0, d)],
                    ssem.at[b],
                )

        def wait_gather(ci, b):
            pltpu.make_async_copy(
                table_hbm.at[idx_v.at[b]], rows_v.at[b], gsem.at[b]
            ).wait()

        def wait_scatter(ci, b):
            g0 = (base + ci * _CHUNK) // 8
            for r in range(_CHUNK // 8):
                pltpu.make_async_copy(
                    rows_v.at[b, pl.ds(r * 8, 8)],
                    out_hbm.at[g0 + r, slice(None), pl.ds(0, d)],
                    ssem.at[b],
                ).wait()

        for b in range(_NBUF):
            start_gather(b, b)

        def body(g, carry):
            for b in range(_NBUF):
                ci = g * _NBUF + b
                wait_gather(ci, b)
                start_scatter(ci, b)
                wait_scatter(ci, b)
                start_gather(ci + _NBUF, b)
            return carry

        lax.fori_loop(0, n_steps - 1, body, 0)

        for b in range(_NBUF):
            ci = (n_steps - 1) * _NBUF + b
            wait_gather(ci, b)
            start_scatter(ci, b)
        for b in range(_NBUF):
            ci = (n_steps - 1) * _NBUF + b
            wait_scatter(ci, b)

    return gather_kernel


def kernel(input, table):
    b, l = input.shape
    vocab, d = table.shape
    n = b * l
    td = _make_depad_transpose(vocab, d)(table.T)
    flat_idx = input.T.reshape(n) * 2
    packed = _make_gather(n, vocab, d)(flat_idx, td.reshape(2 * vocab, d))
    emb = packed.reshape(n, 2 * d)[:, :d].reshape(l, b, d)
    return emb.transpose(1, 0, 2)
